# baseline (speedup 1.0000x reference)
import functools

import jax
import jax.numpy as jnp
from jax import lax
from jax.experimental import pallas as pl
from jax.experimental.pallas import tpu as pltpu
from jax.experimental.pallas import tpu_sc as plsc

N_NODES = 10000
N_EDGES = 320000
D = 128

NC = 2
NS = 16
W = 128
EDGES_PER_TILE = N_EDGES // (NC * NS)
PADDED_EPT = 10240
PAD = PADDED_EPT - EDGES_PER_TILE
WINDOWS_PER_TILE = PADDED_EPT // W
NPH = 5
WPP = WINDOWS_PER_TILE // NPH
ACC_ROWS = N_NODES + 8
ROWCHUNK = 80
N_ROW_CHUNKS = N_NODES // ROWCHUNK
CHUNKS_PER_SUBCORE = -(-N_ROW_CHUNKS // NS)


def _sc_aggregate(x, src2d, dst2d):
    mesh = plsc.VectorSubcoreMesh(core_axis_name="c", subcore_axis_name="s")

    @functools.partial(
        pl.kernel,
        out_type=jax.ShapeDtypeStruct((NC, N_NODES, D), jnp.float32),
        mesh=mesh,
        scratch_types=[
            pltpu.VMEM((WPP, W), jnp.int32),
            pltpu.VMEM((WPP, W), jnp.int32),
            pltpu.VMEM((W, D), jnp.float32),
            pltpu.VMEM((W, D), jnp.float32),
            pltpu.SemaphoreType.DMA,
            pltpu.SemaphoreType.DMA,
            pltpu.VMEM_SHARED((ACC_ROWS, D), jnp.float32),
        ],
    )
    def k(x_hbm, src_hbm, dst_hbm, out_hbm, src_v, dst_v, rows_a, rows_b,
          sem_a, sem_b, acc):
        rows_v = rows_a
        cid = lax.axis_index("c")
        sid = lax.axis_index("s")

        zero16 = jnp.zeros((16,), jnp.float32)

        @pl.loop(0, ROWCHUNK)
        def _(r):
            for c in range(0, D, 16):
                rows_v[r, pl.ds(c, 16)] = zero16

        @pl.loop(0, CHUNKS_PER_SUBCORE)
        def _(j):
            k = sid + j * NS

            @pl.when(k < N_ROW_CHUNKS)
            def _():
                pltpu.sync_copy(rows_v.at[pl.ds(0, ROWCHUNK)],
                                acc.at[pl.ds(k * ROWCHUNK, ROWCHUNK)])

        wid = cid * NS + sid

        plsc.subcore_barrier()

        def gather_start(w, buf, sem):
            pltpu.async_copy(x_hbm.at[src_v.at[w]], buf, sem)

        def gather_wait(w, buf, sem):
            pltpu.make_async_copy(x_hbm.at[src_v.at[w]], buf, sem).wait()

        def scatter_add(w, buf):
            pltpu.sync_copy(buf, acc.at[dst_v.at[w]], add=True)

        @pl.loop(0, NPH)
        def _(p):
            pltpu.sync_copy(src_hbm.at[wid, p], src_v)
            pltpu.sync_copy(dst_hbm.at[wid, p], dst_v)
            gather_start(0, rows_a, sem_a)
            gather_start(1, rows_b, sem_b)

            @pl.loop(0, WPP // 2 - 1)
            def _(h):
                w = h * 2
                gather_wait(w, rows_a, sem_a)
                scatter_add(w, rows_a)
                gather_start(w + 2, rows_a, sem_a)
                gather_wait(w + 1, rows_b, sem_b)
                scatter_add(w + 1, rows_b)
                gather_start(w + 3, rows_b, sem_b)

            gather_wait(WPP - 2, rows_a, sem_a)
            scatter_add(WPP - 2, rows_a)
            gather_wait(WPP - 1, rows_b, sem_b)
            scatter_add(WPP - 1, rows_b)

        plsc.subcore_barrier()

        @pl.loop(0, CHUNKS_PER_SUBCORE)
        def _(j):
            k = sid + j * NS

            @pl.when(k < N_ROW_CHUNKS)
            def _():
                base = k * ROWCHUNK
                pltpu.sync_copy(acc.at[pl.ds(base, ROWCHUNK)],
                                rows_v.at[pl.ds(0, ROWCHUNK)])
                pltpu.sync_copy(rows_v.at[pl.ds(0, ROWCHUNK)],
                                out_hbm.at[cid, pl.ds(base, ROWCHUNK)])

    return k(x, src2d, dst2d)


def _tc_combine(x, partials, W1T, W2T, b):
    BLK = 1000

    def body(x_ref, p_ref, w1_ref, w2_ref, b_ref, o_ref):
        agg = p_ref[0] + p_ref[1]
        o_ref[...] = (
            jnp.dot(x_ref[...], w1_ref[...], preferred_element_type=jnp.float32)
            + jnp.dot(agg, w2_ref[...], preferred_element_type=jnp.float32)
            + b_ref[...]
        )

    return pl.pallas_call(
        body,
        grid=(N_NODES // BLK,),
        in_specs=[
            pl.BlockSpec((BLK, D), lambda i: (i, 0)),
            pl.BlockSpec((NC, BLK, D), lambda i: (0, i, 0)),
            pl.BlockSpec((D, D), lambda i: (0, 0)),
            pl.BlockSpec((D, D), lambda i: (0, 0)),
            pl.BlockSpec((1, D), lambda i: (0, 0)),
        ],
        out_specs=pl.BlockSpec((BLK, D), lambda i: (i, 0)),
        out_shape=jax.ShapeDtypeStruct((N_NODES, D), jnp.float32),
    )(x, partials, W1T, W2T, b)


def kernel(shape_features, edge_index, W1, b1, W2, b2):
    srcr = edge_index[0].reshape(NC * NS, EDGES_PER_TILE)
    dstr = edge_index[1].reshape(NC * NS, EDGES_PER_TILE)
    src_pad = jnp.zeros((NC * NS, PAD), jnp.int32)
    dst_pad = jnp.full((NC * NS, PAD), N_NODES, jnp.int32)
    src2d = jnp.concatenate([srcr, src_pad], axis=1).reshape(NC * NS, NPH, WPP, W)
    dst2d = jnp.concatenate([dstr, dst_pad], axis=1).reshape(NC * NS, NPH, WPP, W)
    partials = _sc_aggregate(shape_features, src2d, dst2d)
    b = (b1 + b2).reshape(1, D)
    return _tc_combine(shape_features, partials, W1.T, W2.T, b)


# speedup vs baseline: 2.9403x; 2.9403x over previous
import functools

import jax
import jax.numpy as jnp
from jax import lax
from jax.experimental import pallas as pl
from jax.experimental.pallas import tpu as pltpu
from jax.experimental.pallas import tpu_sc as plsc

N_NODES = 10000
N_EDGES = 320000
D = 128

NC = 2
NS = 16
W = 80
EDGES_PER_TILE = N_EDGES // (NC * NS)
WINDOWS_PER_TILE = EDGES_PER_TILE // W
NPH = 5
WPP = WINDOWS_PER_TILE // NPH
ROWCHUNK = 80
N_ROW_CHUNKS = N_NODES // ROWCHUNK
CHUNKS_PER_SUBCORE = -(-N_ROW_CHUNKS // NS)


def _sc_aggregate(x, src2d, dst2d):
    mesh = plsc.VectorSubcoreMesh(core_axis_name="c", subcore_axis_name="s")

    @functools.partial(
        pl.kernel,
        out_type=jax.ShapeDtypeStruct((NC, N_NODES, D), jnp.float32),
        mesh=mesh,
        scratch_types=[
            pltpu.VMEM((WPP, W), jnp.int32),
            pltpu.VMEM((WPP, W), jnp.int32),
            pltpu.VMEM((W, D), jnp.float32),
            pltpu.VMEM((W, D), jnp.float32),
            pltpu.VMEM((W, D), jnp.float32),
            pltpu.SemaphoreType.DMA,
            pltpu.SemaphoreType.DMA,
            pltpu.SemaphoreType.DMA,
            pltpu.VMEM_SHARED((N_NODES, D), jnp.float32),
        ],
    )
    def k(x_hbm, src_hbm, dst_hbm, out_hbm, src_v, dst_v, rows_a, rows_b,
          rows_c, sem_a, sem_b, sem_c, acc):
        rows_v = rows_a
        cid = lax.axis_index("c")
        sid = lax.axis_index("s")

        zero16 = jnp.zeros((16,), jnp.float32)

        @pl.loop(0, ROWCHUNK)
        def _(r):
            for c in range(0, D, 16):
                rows_v[r, pl.ds(c, 16)] = zero16

        @pl.loop(0, CHUNKS_PER_SUBCORE)
        def _(j):
            k = sid + j * NS

            @pl.when(k < N_ROW_CHUNKS)
            def _():
                pltpu.sync_copy(rows_v, acc.at[pl.ds(k * ROWCHUNK, ROWCHUNK)])

        wid = cid * NS + sid

        plsc.subcore_barrier()

        def gather_start(w, buf, sem):
            pltpu.async_copy(x_hbm.at[src_v.at[w]], buf, sem)

        def gather_wait(w, buf, sem):
            pltpu.make_async_copy(x_hbm.at[src_v.at[w]], buf, sem).wait()

        def scatter_add(w, buf):
            pltpu.sync_copy(buf, acc.at[dst_v.at[w]], add=True)

        bufs = ((rows_a, sem_a), (rows_b, sem_b), (rows_c, sem_c))

        @pl.loop(0, NPH)
        def _(p):
            pltpu.sync_copy(src_hbm.at[wid, p], src_v)
            pltpu.sync_copy(dst_hbm.at[wid, p], dst_v)
            for i in range(3):
                gather_start(i, *bufs[i])

            @pl.loop(0, WPP // 3)
            def _(h):
                w = h * 3
                for i in range(3):
                    buf, sem = bufs[i]
                    gather_wait(w + i, buf, sem)
                    scatter_add(w + i, buf)

                    @pl.when(w + i + 3 < WPP)
                    def _():
                        gather_start(w + i + 3, buf, sem)

            last = WPP - 1
            buf, sem = bufs[last % 3]
            gather_wait(last, buf, sem)
            scatter_add(last, buf)

        plsc.subcore_barrier()

        @pl.loop(0, CHUNKS_PER_SUBCORE)
        def _(j):
            k = sid + j * NS

            @pl.when(k < N_ROW_CHUNKS)
            def _():
                base = k * ROWCHUNK
                pltpu.sync_copy(acc.at[pl.ds(base, ROWCHUNK)], rows_v)
                pltpu.sync_copy(rows_v, out_hbm.at[cid, pl.ds(base, ROWCHUNK)])

    return k(x, src2d, dst2d)


def _tc_combine(x, partials, W1T, W2T, b):
    BLK = 1000

    def body(x_ref, p_ref, w1_ref, w2_ref, b_ref, o_ref):
        agg = p_ref[0] + p_ref[1]
        o_ref[...] = (
            jnp.dot(x_ref[...], w1_ref[...], preferred_element_type=jnp.float32)
            + jnp.dot(agg, w2_ref[...], preferred_element_type=jnp.float32)
            + b_ref[...]
        )

    return pl.pallas_call(
        body,
        grid=(N_NODES // BLK,),
        in_specs=[
            pl.BlockSpec((BLK, D), lambda i: (i, 0)),
            pl.BlockSpec((NC, BLK, D), lambda i: (0, i, 0)),
            pl.BlockSpec((D, D), lambda i: (0, 0)),
            pl.BlockSpec((D, D), lambda i: (0, 0)),
            pl.BlockSpec((1, D), lambda i: (0, 0)),
        ],
        out_specs=pl.BlockSpec((BLK, D), lambda i: (i, 0)),
        out_shape=jax.ShapeDtypeStruct((N_NODES, D), jnp.float32),
    )(x, partials, W1T, W2T, b)


def kernel(shape_features, edge_index, W1, b1, W2, b2):
    src2d = edge_index[0].reshape(NC * NS, NPH, WPP, W)
    dst2d = edge_index[1].reshape(NC * NS, NPH, WPP, W)
    partials = _sc_aggregate(shape_features, src2d, dst2d)
    b = (b1 + b2).reshape(1, D)
    return _tc_combine(shape_features, partials, W1.T, W2.T, b)
